# baseline (speedup 1.0000x reference)
import functools
import jax
import jax.numpy as jnp
from jax import lax
from jax.experimental import pallas as pl
from jax.experimental.pallas import tpu as pltpu
from jax.experimental.pallas import tpu_sc as plsc

NC = 2
NS = 16
NW = NC * NS
LANES = 16
K = 128
EPS = 1e-5


def _deg_body(CH, NPAD, dst_hbm, out_hbm, dst_v, hist):
    c = lax.axis_index("c")
    s = lax.axis_index("s")
    wid = s * NC + c
    zeros = jnp.zeros((LANES,), jnp.float32)

    @pl.loop(0, NPAD // LANES)
    def _zero(r):
        hist[pl.ds(r * LANES, LANES)] = zeros

    pltpu.sync_copy(dst_hbm.at[wid], dst_v)
    ones = jnp.ones((LANES,), jnp.float32)

    @pl.loop(0, CH)
    def _hist(j):
        for k in range(K // LANES):
            idx = dst_v[j, pl.ds(k * LANES, LANES)]
            plsc.addupdate_scatter(hist, [idx], ones)

    pltpu.sync_copy(hist, out_hbm.at[wid])


def _agg_body(G, CH, RPT, h2_hbm, src_hbm, dst_hbm, out_hbm,
              sidx, didx, gbuf, agg_sh, sem_g, sem_i):
    c = lax.axis_index("c")
    s = lax.axis_index("s")
    wid = s * NC + c
    zeros = jnp.zeros((LANES,), jnp.float32)
    H = gbuf.shape[2]

    @pl.loop(0, K)
    def _zero(r):
        for k in range(H // LANES):
            gbuf[0, r, pl.ds(k * LANES, LANES)] = zeros

    base = s * RPT
    off = 0
    while off < RPT:
        step = min(K, RPT - off)
        pltpu.sync_copy(gbuf.at[0, pl.ds(0, step)],
                        agg_sh.at[pl.ds(base + off, step)])
        off += step

    plsc.subcore_barrier()

    NG = CH // G

    pltpu.async_copy(src_hbm.at[wid, pl.ds(0, G)], sidx.at[0], sem_i)
    pltpu.async_copy(dst_hbm.at[wid, pl.ds(0, G)], didx.at[0], sem_i)

    @pl.loop(0, NG)
    def _grp(g):
        gb = lax.bitwise_and(g, 1)
        pltpu.make_async_copy(src_hbm.at[wid, pl.ds(0, G)],
                              sidx.at[gb], sem_i).wait()
        pltpu.make_async_copy(dst_hbm.at[wid, pl.ds(0, G)],
                              didx.at[gb], sem_i).wait()

        @pl.when(g + 1 < NG)
        def _pref_idx():
            pltpu.async_copy(src_hbm.at[wid, pl.ds((g + 1) * G, G)],
                             sidx.at[1 - gb], sem_i)
            pltpu.async_copy(dst_hbm.at[wid, pl.ds((g + 1) * G, G)],
                             didx.at[1 - gb], sem_i)

        @pl.loop(0, G)
        def _chunk(i):
            pltpu.async_copy(h2_hbm.at[sidx.at[gb, i]], gbuf.at[0],
                             sem_g).wait()
            pltpu.sync_copy(gbuf.at[0], agg_sh.at[didx.at[gb, i]], add=True)

    plsc.subcore_barrier()
    pltpu.sync_copy(agg_sh.at[pl.ds(base, RPT)],
                    out_hbm.at[c, pl.ds(base, RPT)])


def _h2_body(x_ref, w_ref, degp_ref, h2_ref, dinv_ref):
    deg = jnp.sum(degp_ref[...], axis=0) + 1.0
    dinv = lax.rsqrt(deg)
    h = jnp.dot(x_ref[...], w_ref[...], preferred_element_type=jnp.float32)
    h2_ref[...] = h * dinv[:, None]
    dinv_ref[...] = dinv[:, None]


def _bn_body(N, p_ref, h2_ref, dinv_ref, b_ref, gamma_ref, beta_ref, out_ref):
    pre = p_ref[0, :N, :] + p_ref[1, :N, :] + h2_ref[:N, :]
    pre = pre * dinv_ref[:N, :] + b_ref[...][None, :]
    mean = jnp.mean(pre, axis=0)
    var = jnp.mean((pre - mean[None, :]) ** 2, axis=0)
    out = (pre - mean[None, :]) * lax.rsqrt(var + EPS) * gamma_ref[...][None, :]
    out = out + beta_ref[...][None, :]
    out_ref[...] = jnp.maximum(out, 0.0)


def kernel(x, adj_t, W, b, gamma, beta):
    N, D = x.shape
    H = W.shape[1]
    E = adj_t.shape[1]

    G = 16
    CH = -(-E // (NW * K * G)) * G
    E_pad = NW * CH * K
    NPAD = -(-(N + 1) // 1024) * 1024
    RH = NPAD // 128
    RPT = NPAD // NS

    src = adj_t[0]
    dst = adj_t[1]
    pad = E_pad - E
    src2d = jnp.concatenate(
        [src, jnp.zeros((pad,), jnp.int32)]).reshape(NW, CH, K)
    dst2d = jnp.concatenate(
        [dst, jnp.full((pad,), N, jnp.int32)]).reshape(NW, CH, K)
    x_p = jnp.pad(x, ((0, NPAD - N), (0, 0)))

    mesh = plsc.VectorSubcoreMesh(core_axis_name="c", subcore_axis_name="s")

    degp = pl.kernel(
        functools.partial(_deg_body, CH, NPAD),
        out_type=jax.ShapeDtypeStruct((NW, NPAD), jnp.float32),
        mesh=mesh,
        compiler_params=pltpu.CompilerParams(needs_layout_passes=False),
        scratch_types=[
            pltpu.VMEM((CH, K), jnp.int32),
            pltpu.VMEM((NPAD,), jnp.float32),
        ],
    )(dst2d)

    RB = NPAD // 8
    h2, dinv = pl.pallas_call(
        _h2_body,
        grid=(NPAD // RB,),
        in_specs=[
            pl.BlockSpec((RB, D), lambda i: (i, 0)),
            pl.BlockSpec((D, H), lambda i: (0, 0)),
            pl.BlockSpec((NW, RB), lambda i: (0, i)),
        ],
        out_specs=[
            pl.BlockSpec((RB, H), lambda i: (i, 0)),
            pl.BlockSpec((RB, 1), lambda i: (i, 0)),
        ],
        out_shape=[
            jax.ShapeDtypeStruct((NPAD, H), jnp.float32),
            jax.ShapeDtypeStruct((NPAD, 1), jnp.float32),
        ],
    )(x_p, W, degp)

    parts = pl.kernel(
        functools.partial(_agg_body, G, CH, RPT),
        out_type=jax.ShapeDtypeStruct((NC, NPAD, H), jnp.float32),
        mesh=mesh,
        compiler_params=pltpu.CompilerParams(needs_layout_passes=False),
        scratch_types=[
            pltpu.VMEM((2, G, K), jnp.int32),
            pltpu.VMEM((2, G, K), jnp.int32),
            pltpu.VMEM((1, K, H), jnp.float32),
            pltpu.VMEM_SHARED((NPAD, H), jnp.float32),
            pltpu.SemaphoreType.DMA,
            pltpu.SemaphoreType.DMA,
        ],
    )(h2, src2d, dst2d)

    out = pl.pallas_call(
        functools.partial(_bn_body, N),
        out_shape=jax.ShapeDtypeStruct((N, H), jnp.float32),
    )(parts, h2, dinv, b, gamma, beta)
    return out


# speedup vs baseline: 1.0999x; 1.0999x over previous
import functools
import jax
import jax.numpy as jnp
from jax import lax
from jax.experimental import pallas as pl
from jax.experimental.pallas import tpu as pltpu
from jax.experimental.pallas import tpu_sc as plsc

NC = 2
NS = 16
NW = NC * NS
LANES = 16
K = 128
PBITS = 14
EPS = 1e-5


def _deg_body(CH, NPAD, pk_hbm, out_hbm, pk_v, hist):
    c = lax.axis_index("c")
    s = lax.axis_index("s")
    wid = s * NC + c
    zeros = jnp.zeros((LANES,), jnp.float32)

    @pl.loop(0, NPAD // LANES)
    def _zero(r):
        hist[pl.ds(r * LANES, LANES)] = zeros

    pltpu.sync_copy(pk_hbm.at[wid], pk_v)
    ones = jnp.ones((LANES,), jnp.float32)

    @pl.loop(0, CH)
    def _hist(j):
        for k in range(K // LANES):
            v = pk_v[j, pl.ds(k * LANES, LANES)]
            plsc.addupdate_scatter(hist, [lax.shift_right_logical(v, PBITS)],
                                   ones)

    pltpu.sync_copy(hist, out_hbm.at[wid])


def _agg_body(CH, RPT, h2_hbm, pk_hbm, out_hbm,
              pk_v, sbuf, dbuf, gbuf, agg_sh, sem0, sem1):
    c = lax.axis_index("c")
    s = lax.axis_index("s")
    wid = s * NC + c
    zeros = jnp.zeros((LANES,), jnp.float32)
    H = gbuf.shape[2]
    mask = jnp.full((LANES,), (1 << PBITS) - 1, jnp.int32)

    @pl.loop(0, K)
    def _zero(r):
        for k in range(H // LANES):
            gbuf[0, r, pl.ds(k * LANES, LANES)] = zeros

    base = s * RPT
    off = 0
    while off < RPT:
        step = min(K, RPT - off)
        pltpu.sync_copy(gbuf.at[0, pl.ds(0, step)],
                        agg_sh.at[pl.ds(base + off, step)])
        off += step

    pltpu.sync_copy(pk_hbm.at[wid], pk_v)
    plsc.subcore_barrier()

    def unpack(j, slot):
        for k in range(K // LANES):
            v = pk_v[j, pl.ds(k * LANES, LANES)]
            sbuf[slot, pl.ds(k * LANES, LANES)] = lax.bitwise_and(v, mask)
            dbuf[slot, pl.ds(k * LANES, LANES)] = lax.shift_right_logical(
                v, PBITS)

    unpack(0, 0)
    pltpu.async_copy(h2_hbm.at[sbuf.at[0]], gbuf.at[0], sem0)

    @pl.loop(0, CH // 2)
    def _pair(t):
        j0 = 2 * t
        unpack(j0 + 1, 1)
        pltpu.async_copy(h2_hbm.at[sbuf.at[1]], gbuf.at[1], sem1)
        pltpu.make_async_copy(h2_hbm.at[sbuf.at[0]], gbuf.at[0], sem0).wait()
        pltpu.sync_copy(gbuf.at[0], agg_sh.at[dbuf.at[0]], add=True)

        @pl.when(j0 + 2 < CH)
        def _next():
            unpack(j0 + 2, 0)
            pltpu.async_copy(h2_hbm.at[sbuf.at[0]], gbuf.at[0], sem0)

        pltpu.make_async_copy(h2_hbm.at[sbuf.at[1]], gbuf.at[1], sem1).wait()
        pltpu.sync_copy(gbuf.at[1], agg_sh.at[dbuf.at[1]], add=True)

    plsc.subcore_barrier()
    pltpu.sync_copy(agg_sh.at[pl.ds(base, RPT)],
                    out_hbm.at[c, pl.ds(base, RPT)])


def _h2_body(x_ref, w_ref, degp_ref, h2_ref, dinv_ref):
    deg = jnp.sum(degp_ref[...], axis=0) + 1.0
    dinv = lax.rsqrt(deg)
    h = jnp.dot(x_ref[...], w_ref[...], preferred_element_type=jnp.float32)
    h2_ref[...] = h * dinv[:, None]
    dinv_ref[...] = dinv[:, None]


def _bn_body(N, p_ref, h2_ref, dinv_ref, b_ref, gamma_ref, beta_ref, out_ref):
    pre = p_ref[0, :N, :] + p_ref[1, :N, :] + h2_ref[:N, :]
    pre = pre * dinv_ref[:N, :] + b_ref[...][None, :]
    mean = jnp.mean(pre, axis=0)
    var = jnp.mean((pre - mean[None, :]) ** 2, axis=0)
    out = (pre - mean[None, :]) * lax.rsqrt(var + EPS) * gamma_ref[...][None, :]
    out = out + beta_ref[...][None, :]
    out_ref[...] = jnp.maximum(out, 0.0)


def kernel(x, adj_t, W, b, gamma, beta):
    N, D = x.shape
    H = W.shape[1]
    E = adj_t.shape[1]
    assert N < (1 << PBITS)

    CH = -(-E // (NW * K * 2)) * 2
    E_pad = NW * CH * K
    NPAD = -(-(N + 1) // 1024) * 1024
    RPT = NPAD // NS

    src = adj_t[0]
    dst = adj_t[1]
    pad = E_pad - E
    packed = jnp.bitwise_or(src, jnp.left_shift(dst, PBITS))
    pk2d = jnp.concatenate(
        [packed, jnp.full((pad,), N << PBITS, jnp.int32)]).reshape(NW, CH, K)
    x_p = jnp.pad(x, ((0, NPAD - N), (0, 0)))

    mesh = plsc.VectorSubcoreMesh(core_axis_name="c", subcore_axis_name="s")

    degp = pl.kernel(
        functools.partial(_deg_body, CH, NPAD),
        out_type=jax.ShapeDtypeStruct((NW, NPAD), jnp.float32),
        mesh=mesh,
        compiler_params=pltpu.CompilerParams(needs_layout_passes=False),
        scratch_types=[
            pltpu.VMEM((CH, K), jnp.int32),
            pltpu.VMEM((NPAD,), jnp.float32),
        ],
    )(pk2d)

    RB = NPAD // 8
    h2, dinv = pl.pallas_call(
        _h2_body,
        grid=(NPAD // RB,),
        in_specs=[
            pl.BlockSpec((RB, D), lambda i: (i, 0)),
            pl.BlockSpec((D, H), lambda i: (0, 0)),
            pl.BlockSpec((NW, RB), lambda i: (0, i)),
        ],
        out_specs=[
            pl.BlockSpec((RB, H), lambda i: (i, 0)),
            pl.BlockSpec((RB, 1), lambda i: (i, 0)),
        ],
        out_shape=[
            jax.ShapeDtypeStruct((NPAD, H), jnp.float32),
            jax.ShapeDtypeStruct((NPAD, 1), jnp.float32),
        ],
    )(x_p, W, degp)

    parts = pl.kernel(
        functools.partial(_agg_body, CH, RPT),
        out_type=jax.ShapeDtypeStruct((NC, NPAD, H), jnp.float32),
        mesh=mesh,
        compiler_params=pltpu.CompilerParams(needs_layout_passes=False),
        scratch_types=[
            pltpu.VMEM((CH, K), jnp.int32),
            pltpu.VMEM((2, K), jnp.int32),
            pltpu.VMEM((2, K), jnp.int32),
            pltpu.VMEM((2, K, H), jnp.float32),
            pltpu.VMEM_SHARED((NPAD, H), jnp.float32),
            pltpu.SemaphoreType.DMA,
            pltpu.SemaphoreType.DMA,
        ],
    )(h2, pk2d)

    out = pl.pallas_call(
        functools.partial(_bn_body, N),
        out_shape=jax.ShapeDtypeStruct((N, H), jnp.float32),
    )(parts, h2, dinv, b, gamma, beta)
    return out
